# Initial kernel scaffold; baseline (speedup 1.0000x reference)
import jax
import jax.numpy as jnp
from jax.experimental import pallas as pl


def kernel(x, edge_index, edge_attr, batch, experimental_feat, params):
    raise NotImplementedError("write your pallas kernel here")


# speedup vs baseline: 2.6550x; 2.6550x over previous
import functools

import jax
import jax.numpy as jnp
from jax import lax
from jax.experimental import pallas as pl
from jax.experimental.pallas import tpu as pltpu
from jax.experimental.pallas import tpu_sc as plsc

N = 10000
E = 320000
D = 128
H = 128
EE = 64
G = 512
XD = 64
L = 5
NUM_BOND_TYPE = 6
NUM_BOND_DIR = 4
NUM_CLS = NUM_BOND_TYPE * NUM_BOND_DIR

NC = 2
NS = 16
NW = NC * NS
R = 320
N_PAD = R * NW
K = 256
NCHUNK = (E + K - 1) // K


def _mlp2_body(x_ref, w1_ref, b1_ref, w2_ref, b2_ref, o_ref):
    h = jnp.maximum(jnp.dot(x_ref[...], w1_ref[...],
                            preferred_element_type=jnp.float32) + b1_ref[...], 0.0)
    o_ref[...] = jnp.dot(h, w2_ref[...],
                         preferred_element_type=jnp.float32) + b2_ref[...]


def _mlp2(x, W1, b1, W2, b2):
    return pl.pallas_call(
        _mlp2_body,
        out_shape=jax.ShapeDtypeStruct((x.shape[0], W2.shape[1]), jnp.float32),
    )(x, W1, b1.reshape(1, -1), W2, b2.reshape(1, -1))


def _layer_body(h_ref, agg_ref, w1_ref, b1_ref, w2_ref, b2_ref, g_ref, be_ref,
                o_ref):
    z = h_ref[...] + agg_ref[...]
    y = jnp.maximum(jnp.dot(z, w1_ref[...], preferred_element_type=jnp.float32)
                    + b1_ref[...], 0.0)
    y = jnp.dot(y, w2_ref[...], preferred_element_type=jnp.float32) + b2_ref[...]
    mu = jnp.mean(y, axis=0, keepdims=True)
    var = jnp.mean((y - mu) ** 2, axis=0, keepdims=True)
    yn = (y - mu) * jax.lax.rsqrt(var + 1e-5) * g_ref[...] + be_ref[...]
    o_ref[...] = jnp.maximum(yn, 0.0) + h_ref[...]


def _layer_mlp(h, agg, Wm1, bm1, Wm2, bm2, gamma, beta):
    return pl.pallas_call(
        _layer_body,
        out_shape=jax.ShapeDtypeStruct((h.shape[0], H), jnp.float32),
    )(h, agg, Wm1, bm1.reshape(1, -1), Wm2, bm2.reshape(1, -1),
      gamma.reshape(1, -1), beta.reshape(1, -1))


def _head_body(gm_ref, gx_ref, ex_ref, we1_ref, be1_ref, we2_ref, be2_ref,
               wh1a_ref, wh1b_ref, wh1c_ref, bh1_ref, wh2_ref, bh2_ref,
               wh3_ref, bh3_ref, out_ref, gemb_ref, comb_ref):
    gm = gm_ref[...]
    gx = gx_ref[...]
    ex = jnp.maximum(jnp.dot(ex_ref[...], we1_ref[...],
                             preferred_element_type=jnp.float32) + be1_ref[...], 0.0)
    ex = jnp.maximum(jnp.dot(ex, we2_ref[...],
                             preferred_element_type=jnp.float32) + be2_ref[...], 0.0)
    gemb_ref[...] = jnp.concatenate([gm, gx], axis=1)
    comb_ref[...] = jnp.concatenate([gm, gx, ex], axis=1)
    o = (jnp.dot(gm, wh1a_ref[...], preferred_element_type=jnp.float32)
         + jnp.dot(gx, wh1b_ref[...], preferred_element_type=jnp.float32)
         + jnp.dot(ex, wh1c_ref[...], preferred_element_type=jnp.float32)
         + bh1_ref[...])
    o = jnp.maximum(o, 0.0)
    o = jnp.maximum(jnp.dot(o, wh2_ref[...], preferred_element_type=jnp.float32)
                    + bh2_ref[...], 0.0)
    out_ref[...] = jnp.dot(o, wh3_ref[...],
                           preferred_element_type=jnp.float32) + bh3_ref[...]


def _head(h_mean, h_max, exf, p):
    Wh1 = p['Wh1']
    return pl.pallas_call(
        _head_body,
        out_shape=(
            jax.ShapeDtypeStruct((G, 1), jnp.float32),
            jax.ShapeDtypeStruct((G, 2 * H), jnp.float32),
            jax.ShapeDtypeStruct((G, 2 * H + 128), jnp.float32),
        ),
    )(h_mean, h_max, exf, p['We1'], p['be1'].reshape(1, -1),
      p['We2'], p['be2'].reshape(1, -1),
      Wh1[:H], Wh1[H:2 * H], Wh1[2 * H:], p['bh1'].reshape(1, -1),
      p['Wh2'], p['bh2'].reshape(1, -1), p['Wh3'], p['bh3'].reshape(1, -1))


_SC_MESH = plsc.VectorSubcoreMesh(core_axis_name="c", subcore_axis_name="s")


def _edge_agg_body(h_hbm, src_hbm, dst_hbm, cls_hbm, estart_hbm, c_hbm,
                   agg_hbm,
                   acc_v, rows_v, srcidx_v, dst_v, cls_v, estart_v, c_v, sem):
    wid = lax.axis_index("s") * NC + lax.axis_index("c")
    row0 = wid * R

    pltpu.sync_copy(c_hbm, c_v)
    pltpu.sync_copy(estart_hbm, estart_v)
    ev = estart_v[pl.ds(wid, 16)]
    e0 = ev[0]
    e1 = ev[1]

    def _zero(r, _):
        for f in range(H // 16):
            acc_v[r, pl.ds(f * 16, 16)] = jnp.zeros((16,), jnp.float32)
        return 0
    lax.fori_loop(0, R + 1, _zero, 0)

    c0 = e0 >> 8
    c1 = (e1 + (K - 1)) >> 8

    def _chunk(ci, _):
        base = ci * K
        pltpu.sync_copy(src_hbm.at[pl.ds(base, K)], srcidx_v)
        gather = pltpu.async_copy(h_hbm.at[srcidx_v], rows_v, sem)
        pltpu.sync_copy(dst_hbm.at[pl.ds(base, K)], dst_v)
        pltpu.sync_copy(cls_hbm.at[pl.ds(base, K)], cls_v)
        gather.wait()

        def _group(gi, _):
            goff = gi * 16
            gbase = base + goff
            g16 = gbase + lax.iota(jnp.int32, 16)
            valid = jnp.logical_and(g16 >= e0, g16 < e1)
            d16 = jnp.where(valid, dst_v[pl.ds(goff, 16)] - row0, R)
            c16 = cls_v[pl.ds(goff, 16)]
            for i in range(16):
                d = d16[i]
                c = c16[i]
                j = goff + i
                for f in range(H // 16):
                    v = (rows_v[j, pl.ds(f * 16, 16)]
                         + c_v[c, pl.ds(f * 16, 16)])
                    plsc.addupdate(acc_v.at[d, pl.ds(f * 16, 16)],
                                   jnp.maximum(v, 0.0))
            return 0
        lax.fori_loop(0, K // 16, _group, 0)
        return 0

    lax.fori_loop(c0, c1, _chunk, 0)

    pltpu.sync_copy(acc_v.at[pl.ds(0, R)], agg_hbm.at[pl.ds(row0, R)])


_edge_agg_call = functools.partial(
    pl.kernel,
    _edge_agg_body,
    out_type=jax.ShapeDtypeStruct((N_PAD, H), jnp.float32),
    mesh=_SC_MESH,
    scratch_types=[
        pltpu.VMEM((R + 1, H), jnp.float32),
        pltpu.VMEM((K, H), jnp.float32),
        pltpu.VMEM((K,), jnp.int32),
        pltpu.VMEM((K,), jnp.int32),
        pltpu.VMEM((K,), jnp.int32),
        pltpu.VMEM((NW + 16,), jnp.int32),
        pltpu.VMEM((NUM_CLS, H), jnp.float32),
        pltpu.SemaphoreType.DMA,
    ],
)()


def _edge_agg(h, s_src, s_dst, s_cls, estart, C):
    return _edge_agg_call(h, s_src, s_dst, s_cls, estart, C)[:N]


def kernel(x, edge_index, edge_attr, batch, experimental_feat, params):
    p = params
    h = _mlp2(x, p['W1'], p['b1'], p['W2'], p['b2'])

    bt = jnp.clip(edge_attr[:, 0], 0, NUM_BOND_TYPE - 1)
    bd = jnp.clip(edge_attr[:, 1], 0, NUM_BOND_DIR - 1)
    cls = (bt * NUM_BOND_DIR + bd).astype(jnp.int32)
    src = edge_index[0].astype(jnp.int32)
    dst = edge_index[1].astype(jnp.int32)

    perm = jnp.argsort(dst)
    s_src = src[perm]
    s_dst = dst[perm]
    s_cls = cls[perm]
    bounds = (jnp.arange(NW + 1, dtype=jnp.int32) * R).clip(max=N)
    estart = jnp.searchsorted(s_dst, bounds).astype(jnp.int32)
    estart = jnp.pad(estart, (0, NW + 16 - estart.shape[0]))

    eemb_cls = (p['et'][:, None, :] + p['ed'][None, :, :]).reshape(NUM_CLS, EE)

    for lp in p['layers']:
        C = eemb_cls @ lp['lin_e'] + lp['lin_eb']
        agg = _edge_agg(h, s_src, s_dst, s_cls, estart, C)
        h = _layer_mlp(h, agg, lp['Wm1'], lp['bm1'], lp['Wm2'], lp['bm2'],
                       lp['gamma'], lp['beta'])

    ones = jnp.ones((N,), jnp.float32)
    cnt = jax.ops.segment_sum(ones, batch, num_segments=G)
    h_mean = jax.ops.segment_sum(h, batch, num_segments=G) / jnp.clip(cnt, 1.0)[:, None]
    h_max = jax.ops.segment_max(h, batch, num_segments=G)
    h_max = jnp.where(jnp.isfinite(h_max), h_max, 0.0)
    out, gemb, comb = _head(h_mean, h_max, experimental_feat, p)
    return (out, gemb, comb)


# speedup vs baseline: 2.7083x; 1.0201x over previous
import functools

import jax
import jax.numpy as jnp
from jax import lax
from jax.experimental import pallas as pl
from jax.experimental.pallas import tpu as pltpu
from jax.experimental.pallas import tpu_sc as plsc

N = 10000
E = 320000
D = 128
H = 128
EE = 64
G = 512
XD = 64
L = 5
NUM_BOND_TYPE = 6
NUM_BOND_DIR = 4
NUM_CLS = NUM_BOND_TYPE * NUM_BOND_DIR

NC = 2
NS = 16
NW = NC * NS
R = 320
N_PAD = R * NW
K = 256
NCHUNK = (E + K - 1) // K


def _mlp2_body(x_ref, w1_ref, b1_ref, w2_ref, b2_ref, o_ref):
    h = jnp.maximum(jnp.dot(x_ref[...], w1_ref[...],
                            preferred_element_type=jnp.float32) + b1_ref[...], 0.0)
    o_ref[...] = jnp.dot(h, w2_ref[...],
                         preferred_element_type=jnp.float32) + b2_ref[...]


def _mlp2(x, W1, b1, W2, b2):
    return pl.pallas_call(
        _mlp2_body,
        out_shape=jax.ShapeDtypeStruct((x.shape[0], W2.shape[1]), jnp.float32),
    )(x, W1, b1.reshape(1, -1), W2, b2.reshape(1, -1))


def _layer_body(h_ref, agg_ref, w1_ref, b1_ref, w2_ref, b2_ref, g_ref, be_ref,
                o_ref):
    z = h_ref[...] + agg_ref[...]
    y = jnp.maximum(jnp.dot(z, w1_ref[...], preferred_element_type=jnp.float32)
                    + b1_ref[...], 0.0)
    y = jnp.dot(y, w2_ref[...], preferred_element_type=jnp.float32) + b2_ref[...]
    mu = jnp.mean(y, axis=0, keepdims=True)
    var = jnp.mean((y - mu) ** 2, axis=0, keepdims=True)
    yn = (y - mu) * jax.lax.rsqrt(var + 1e-5) * g_ref[...] + be_ref[...]
    o_ref[...] = jnp.maximum(yn, 0.0) + h_ref[...]


def _layer_mlp(h, agg, Wm1, bm1, Wm2, bm2, gamma, beta):
    return pl.pallas_call(
        _layer_body,
        out_shape=jax.ShapeDtypeStruct((h.shape[0], H), jnp.float32),
    )(h, agg, Wm1, bm1.reshape(1, -1), Wm2, bm2.reshape(1, -1),
      gamma.reshape(1, -1), beta.reshape(1, -1))


def _head_body(gm_ref, gx_ref, ex_ref, we1_ref, be1_ref, we2_ref, be2_ref,
               wh1a_ref, wh1b_ref, wh1c_ref, bh1_ref, wh2_ref, bh2_ref,
               wh3_ref, bh3_ref, out_ref, gemb_ref, comb_ref):
    gm = gm_ref[...]
    gx = gx_ref[...]
    ex = jnp.maximum(jnp.dot(ex_ref[...], we1_ref[...],
                             preferred_element_type=jnp.float32) + be1_ref[...], 0.0)
    ex = jnp.maximum(jnp.dot(ex, we2_ref[...],
                             preferred_element_type=jnp.float32) + be2_ref[...], 0.0)
    gemb_ref[...] = jnp.concatenate([gm, gx], axis=1)
    comb_ref[...] = jnp.concatenate([gm, gx, ex], axis=1)
    o = (jnp.dot(gm, wh1a_ref[...], preferred_element_type=jnp.float32)
         + jnp.dot(gx, wh1b_ref[...], preferred_element_type=jnp.float32)
         + jnp.dot(ex, wh1c_ref[...], preferred_element_type=jnp.float32)
         + bh1_ref[...])
    o = jnp.maximum(o, 0.0)
    o = jnp.maximum(jnp.dot(o, wh2_ref[...], preferred_element_type=jnp.float32)
                    + bh2_ref[...], 0.0)
    out_ref[...] = jnp.dot(o, wh3_ref[...],
                           preferred_element_type=jnp.float32) + bh3_ref[...]


def _head(h_mean, h_max, exf, p):
    Wh1 = p['Wh1']
    return pl.pallas_call(
        _head_body,
        out_shape=(
            jax.ShapeDtypeStruct((G, 1), jnp.float32),
            jax.ShapeDtypeStruct((G, 2 * H), jnp.float32),
            jax.ShapeDtypeStruct((G, 2 * H + 128), jnp.float32),
        ),
    )(h_mean, h_max, exf, p['We1'], p['be1'].reshape(1, -1),
      p['We2'], p['be2'].reshape(1, -1),
      Wh1[:H], Wh1[H:2 * H], Wh1[2 * H:], p['bh1'].reshape(1, -1),
      p['Wh2'], p['bh2'].reshape(1, -1), p['Wh3'], p['bh3'].reshape(1, -1))


_SC_MESH = plsc.VectorSubcoreMesh(core_axis_name="c", subcore_axis_name="s")


def _edge_agg_body(h_hbm, src_hbm, dc_hbm, estart_hbm, c_hbm,
                   agg_hbm,
                   acc_v, rows_a, rows_b, src_a, src_b, dc_a, dc_b,
                   estart_v, c_v, sem_a, sem_b):
    wid = lax.axis_index("s") * NC + lax.axis_index("c")
    row0 = wid * R

    pltpu.sync_copy(c_hbm, c_v)
    pltpu.sync_copy(estart_hbm, estart_v)
    ev = estart_v[pl.ds(wid, 16)]
    e0 = ev[0]
    e1 = ev[1]

    def _zero(r, _):
        for f in range(H // 16):
            acc_v[r, pl.ds(f * 16, 16)] = jnp.zeros((16,), jnp.float32)
        return 0
    lax.fori_loop(0, R + 1, _zero, 0)

    c0 = e0 >> 8
    c1 = (e1 + (K - 1)) >> 8

    def _load(ci, src_v, dc_v, rows_v, sem):
        cieff = jnp.minimum(ci, NCHUNK - 1)
        pltpu.sync_copy(src_hbm.at[pl.ds(cieff * K, K)], src_v)
        pltpu.sync_copy(dc_hbm.at[pl.ds(cieff * 2 * K, 2 * K)], dc_v)
        return pltpu.async_copy(h_hbm.at[src_v], rows_v, sem)

    def _compute(ci, dc_v, rows_v):
        base = jnp.minimum(ci, NCHUNK - 1) * K
        e1m = jnp.where(ci < c1, e1, jnp.int32(-1))

        def _group(gi, _):
            goff = gi * 16
            g16 = base + goff + lax.iota(jnp.int32, 16)
            valid = jnp.logical_and(g16 >= e0, g16 < e1m)
            d16 = jnp.where(valid, dc_v[pl.ds(goff, 16)] - row0, R)
            c16 = dc_v[pl.ds(K + goff, 16)]
            for i in range(16):
                d = d16[i]
                c = c16[i]
                j = goff + i
                for f in range(H // 16):
                    v = (rows_v[j, pl.ds(f * 16, 16)]
                         + c_v[c, pl.ds(f * 16, 16)])
                    plsc.addupdate(acc_v.at[d, pl.ds(f * 16, 16)],
                                   jnp.maximum(v, 0.0))
            return 0
        lax.fori_loop(0, K // 16, _group, 0)

    def _pair(pi, _):
        ci0 = c0 + 2 * pi
        ci1 = ci0 + 1
        g_a = _load(ci0, src_a, dc_a, rows_a, sem_a)
        g_a.wait()
        g_b = _load(ci1, src_b, dc_b, rows_b, sem_b)
        _compute(ci0, dc_a, rows_a)
        g_b.wait()
        _compute(ci1, dc_b, rows_b)
        return 0

    npairs = (c1 - c0 + 1) >> 1
    lax.fori_loop(0, npairs, _pair, 0)

    pltpu.sync_copy(acc_v.at[pl.ds(0, R)], agg_hbm.at[pl.ds(row0, R)])


_edge_agg_call = functools.partial(
    pl.kernel,
    _edge_agg_body,
    out_type=jax.ShapeDtypeStruct((N_PAD, H), jnp.float32),
    mesh=_SC_MESH,
    scratch_types=[
        pltpu.VMEM((R + 1, H), jnp.float32),
        pltpu.VMEM((K, H), jnp.float32),
        pltpu.VMEM((K, H), jnp.float32),
        pltpu.VMEM((K,), jnp.int32),
        pltpu.VMEM((K,), jnp.int32),
        pltpu.VMEM((2 * K,), jnp.int32),
        pltpu.VMEM((2 * K,), jnp.int32),
        pltpu.VMEM((NW + 16,), jnp.int32),
        pltpu.VMEM((NUM_CLS, H), jnp.float32),
        pltpu.SemaphoreType.DMA,
        pltpu.SemaphoreType.DMA,
    ],
)()


def _edge_agg(h, s_src, s_dc, estart, C):
    return _edge_agg_call(h, s_src, s_dc, estart, C)[:N]


def kernel(x, edge_index, edge_attr, batch, experimental_feat, params):
    p = params
    h = _mlp2(x, p['W1'], p['b1'], p['W2'], p['b2'])

    bt = jnp.clip(edge_attr[:, 0], 0, NUM_BOND_TYPE - 1)
    bd = jnp.clip(edge_attr[:, 1], 0, NUM_BOND_DIR - 1)
    cls = (bt * NUM_BOND_DIR + bd).astype(jnp.int32)
    src = edge_index[0].astype(jnp.int32)
    dst = edge_index[1].astype(jnp.int32)

    perm = jnp.argsort(dst)
    s_src = src[perm]
    s_dst = dst[perm]
    s_cls = cls[perm]
    bounds = (jnp.arange(NW + 1, dtype=jnp.int32) * R).clip(max=N)
    estart = jnp.searchsorted(s_dst, bounds).astype(jnp.int32)
    estart = jnp.pad(estart, (0, NW + 16 - estart.shape[0]))
    s_dc = jnp.concatenate([s_dst.reshape(NCHUNK, K),
                            s_cls.reshape(NCHUNK, K)], axis=1).reshape(-1)

    eemb_cls = (p['et'][:, None, :] + p['ed'][None, :, :]).reshape(NUM_CLS, EE)

    for lp in p['layers']:
        C = eemb_cls @ lp['lin_e'] + lp['lin_eb']
        agg = _edge_agg(h, s_src, s_dc, estart, C)
        h = _layer_mlp(h, agg, lp['Wm1'], lp['bm1'], lp['Wm2'], lp['bm2'],
                       lp['gamma'], lp['beta'])

    ones = jnp.ones((N,), jnp.float32)
    cnt = jax.ops.segment_sum(ones, batch, num_segments=G)
    h_mean = jax.ops.segment_sum(h, batch, num_segments=G) / jnp.clip(cnt, 1.0)[:, None]
    h_max = jax.ops.segment_max(h, batch, num_segments=G)
    h_max = jnp.where(jnp.isfinite(h_max), h_max, 0.0)
    out, gemb, comb = _head(h_mean, h_max, experimental_feat, p)
    return (out, gemb, comb)
